# Initial kernel scaffold; baseline (speedup 1.0000x reference)
import jax
import jax.numpy as jnp
from jax.experimental import pallas as pl


def kernel(params, objs, triples):
    raise NotImplementedError("write your pallas kernel here")


# speedup vs baseline: 2.0582x; 2.0582x over previous
import functools

import jax
import jax.numpy as jnp
from jax import lax
from jax.experimental import pallas as pl
from jax.experimental.pallas import tpu as pltpu
from jax.experimental.pallas import tpu_sc as plsc

N = 10000
E = 160000
D = 64
TW = 2 * D
NOBJ = 101
NPRED = 46
NC, NSC = 2, 16
NW = NC * NSC
CH = 128
ECH = E // CH
EIT = (ECH + NW - 1) // NW
NPAD = 10112
NCHN = NPAD // CH
NITN = (NCHN + NW - 1) // NW
OUTW = 10
RPW = N // OUTW
STG = 40
NSTG = RPW // STG

BLK_E = 2000
BLK_N = 2000

_MESH = plsc.VectorSubcoreMesh(
    core_axis_name="c", subcore_axis_name="s",
    num_cores=NC, num_subcores=NSC)

_f32 = jnp.float32


@functools.partial(
    pl.kernel,
    out_type=[jax.ShapeDtypeStruct((NPAD, TW), _f32),
              jax.ShapeDtypeStruct((E, TW), _f32),
              jax.ShapeDtypeStruct((NC, N, TW), _f32)],
    mesh=_MESH,
    scratch_types=[pltpu.VMEM((CH,), jnp.int32),
                   pltpu.VMEM((CH,), jnp.int32),
                   pltpu.VMEM((CH, TW), _f32),
                   pltpu.VMEM((CH, TW), _f32),
                   pltpu.VMEM((STG, TW), _f32),
                   pltpu.VMEM_SHARED((N, TW), _f32),
                   pltpu.SemaphoreType.DMA],
)
def _sc_init(t0, tbp, objs_r, p_r, s_r, o_r, zeros_in, ones_in,
             ttab, p0, cnts,
             idx_a, idx_b, rows_a, ones_v, zstg, acc, sem_a):
    cid = lax.axis_index("c")
    sid = lax.axis_index("s")
    wid = sid * NC + cid

    pltpu.sync_copy(ones_in, ones_v)

    @pl.when(sid < OUTW)
    def _zero():
        pltpu.sync_copy(zeros_in.at[pl.ds(0, STG)], zstg)
        for j in range(NSTG):
            pltpu.sync_copy(zstg, acc.at[pl.ds(sid * RPW + j * STG, STG)])

    def node_it(k, carry):
        c = wid + NW * k

        @pl.when(c < NCHN)
        def _():
            base = c * CH
            pltpu.sync_copy(objs_r.at[pl.ds(base, CH)], idx_a)
            pltpu.async_copy(t0.at[idx_a], rows_a, sem_a).wait()
            pltpu.sync_copy(rows_a, ttab.at[pl.ds(base, CH)])
        return carry

    lax.fori_loop(0, NITN, node_it, 0)

    plsc.subcore_barrier()

    def edge_it(k, carry):
        c = wid + NW * k

        @pl.when(c < ECH)
        def _():
            base = c * CH
            pltpu.sync_copy(p_r.at[pl.ds(base, CH)], idx_a)
            pltpu.async_copy(tbp.at[idx_a], rows_a, sem_a).wait()
            pltpu.sync_copy(rows_a, p0.at[pl.ds(base, CH)])
            pltpu.sync_copy(s_r.at[pl.ds(base, CH)], idx_a)
            pltpu.sync_copy(o_r.at[pl.ds(base, CH)], idx_b)
            pltpu.sync_copy(ones_v, acc.at[idx_a], add=True)
            pltpu.sync_copy(ones_v, acc.at[idx_b], add=True)
        return carry

    lax.fori_loop(0, EIT, edge_it, 0)

    plsc.subcore_barrier()

    @pl.when(sid < OUTW)
    def _out():
        for j in range(NSTG):
            pltpu.sync_copy(acc.at[pl.ds(sid * RPW + j * STG, STG)], zstg)
            pltpu.sync_copy(zstg, cnts.at[cid, pl.ds(sid * RPW + j * STG, STG)])


@functools.partial(
    pl.kernel,
    out_type=[jax.ShapeDtypeStruct((E, TW), _f32),
              jax.ShapeDtypeStruct((E, TW), _f32)],
    mesh=_MESH,
    scratch_types=[pltpu.VMEM((CH,), jnp.int32),
                   pltpu.VMEM((CH,), jnp.int32),
                   pltpu.VMEM((CH, TW), _f32),
                   pltpu.VMEM((CH, TW), _f32),
                   pltpu.SemaphoreType.DMA,
                   pltpu.SemaphoreType.DMA],
)
def _sc_gather(tt, s_r, o_r, ga, gb, idx_a, idx_b, rows_a, rows_b,
               sem_a, sem_b):
    cid = lax.axis_index("c")
    sid = lax.axis_index("s")
    wid = sid * NC + cid

    def it(k, carry):
        c = wid + NW * k

        @pl.when(c < ECH)
        def _():
            base = c * CH
            pltpu.sync_copy(s_r.at[pl.ds(base, CH)], idx_a)
            pltpu.sync_copy(o_r.at[pl.ds(base, CH)], idx_b)
            cpa = pltpu.async_copy(tt.at[idx_a], rows_a, sem_a)
            cpb = pltpu.async_copy(tt.at[idx_b], rows_b, sem_b)
            cpa.wait()
            cpb.wait()
            pltpu.sync_copy(rows_a, ga.at[pl.ds(base, CH)])
            pltpu.sync_copy(rows_b, gb.at[pl.ds(base, CH)])
        return carry

    lax.fori_loop(0, EIT, it, 0)


@functools.partial(
    pl.kernel,
    out_type=jax.ShapeDtypeStruct((NC, N, TW), _f32),
    mesh=_MESH,
    scratch_types=[pltpu.VMEM((CH,), jnp.int32),
                   pltpu.VMEM((CH,), jnp.int32),
                   pltpu.VMEM((CH, TW), _f32),
                   pltpu.VMEM((CH, TW), _f32),
                   pltpu.VMEM((STG, TW), _f32),
                   pltpu.VMEM_SHARED((N, TW), _f32)],
)
def _sc_scatter(vs, vo, s_r, o_r, zeros_in, pooled,
                idx_a, idx_b, rows_a, rows_b, zstage, acc):
    cid = lax.axis_index("c")
    sid = lax.axis_index("s")
    wid = sid * NC + cid

    @pl.when(sid < OUTW)
    def _zero():
        pltpu.sync_copy(zeros_in.at[pl.ds(0, STG)], zstage)
        for j in range(NSTG):
            pltpu.sync_copy(zstage, acc.at[pl.ds(sid * RPW + j * STG, STG)])

    plsc.subcore_barrier()

    def it(k, carry):
        c = wid + NW * k

        @pl.when(c < ECH)
        def _():
            base = c * CH
            pltpu.sync_copy(s_r.at[pl.ds(base, CH)], idx_a)
            pltpu.sync_copy(o_r.at[pl.ds(base, CH)], idx_b)
            pltpu.sync_copy(vs.at[pl.ds(base, CH)], rows_a)
            pltpu.sync_copy(vo.at[pl.ds(base, CH)], rows_b)
            pltpu.sync_copy(rows_a, acc.at[idx_a], add=True)
            pltpu.sync_copy(rows_b, acc.at[idx_b], add=True)
        return carry

    lax.fori_loop(0, EIT, it, 0)

    plsc.subcore_barrier()

    @pl.when(sid < OUTW)
    def _out():
        for j in range(NSTG):
            pltpu.sync_copy(acc.at[pl.ds(sid * RPW + j * STG, STG)], zstage)
            pltpu.sync_copy(zstage, pooled.at[cid, pl.ds(sid * RPW + j * STG, STG)])


def _tc_init(obj_emb, pred_emb, a0, c0, b0):
    def body(oe, pe, a, c, b, t0, tbp):
        t0[:] = jnp.concatenate(
            [jnp.dot(oe[:], a[:], preferred_element_type=_f32, precision=lax.Precision.HIGHEST),
             jnp.dot(oe[:], c[:], preferred_element_type=_f32, precision=lax.Precision.HIGHEST)], axis=1)
        tbp[:] = jnp.concatenate(
            [jnp.dot(pe[:], b[:], preferred_element_type=_f32, precision=lax.Precision.HIGHEST),
             jnp.zeros((NPRED, D), _f32)], axis=1)

    return pl.pallas_call(
        body,
        out_shape=[jax.ShapeDtypeStruct((NOBJ, TW), _f32),
                   jax.ShapeDtypeStruct((NPRED, TW), _f32)],
    )(obj_emb, pred_emb, a0, c0, b0)


def _tc_edge(ga, gb, pv, bmat, b1, w2, b2, first):
    def body(ga_r, gb_r, pv_r, *rest):
        pre = ga_r[:, :D] + gb_r[:, D:]
        if first:
            b1_r, w2_r, b2_r, os_r, op_r, oo_r = rest
            pre = pre + pv_r[:, :D]
        else:
            bm_r, b1_r, w2_r, b2_r, os_r, op_r, oo_r = rest
            pre = pre + jnp.dot(pv_r[:], bm_r[:], preferred_element_type=_f32, precision=lax.Precision.HIGHEST)
        h = jnp.maximum(pre + b1_r[:], 0.0)
        t = jnp.maximum(
            jnp.dot(h, w2_r[:], preferred_element_type=_f32, precision=lax.Precision.HIGHEST) + b2_r[:], 0.0)
        z = jnp.zeros((t.shape[0], D), _f32)
        os_r[:] = jnp.concatenate([t[:, :D], z], axis=1)
        op_r[:] = t[:, D:2 * D]
        oo_r[:] = jnp.concatenate([z, t[:, 2 * D:]], axis=1)

    row = lambda i: (i, 0)
    zero = lambda i: (0, 0)
    in_specs = [pl.BlockSpec((BLK_E, TW), row),
                pl.BlockSpec((BLK_E, TW), row)]
    args = [ga, gb]
    if first:
        in_specs.append(pl.BlockSpec((BLK_E, TW), row))
    else:
        in_specs.append(pl.BlockSpec((BLK_E, D), row))
    args.append(pv)
    if not first:
        in_specs.append(pl.BlockSpec((D, D), zero))
        args.append(bmat)
    in_specs += [pl.BlockSpec((1, D), zero),
                 pl.BlockSpec((D, 3 * D), zero),
                 pl.BlockSpec((1, 3 * D), zero)]
    args += [b1.reshape(1, D), w2, b2.reshape(1, 3 * D)]
    return pl.pallas_call(
        body,
        grid=(E // BLK_E,),
        in_specs=in_specs,
        out_specs=[pl.BlockSpec((BLK_E, TW), row),
                   pl.BlockSpec((BLK_E, D), row),
                   pl.BlockSpec((BLK_E, TW), row)],
        out_shape=[jax.ShapeDtypeStruct((E, TW), _f32),
                   jax.ShapeDtypeStruct((E, D), _f32),
                   jax.ShapeDtypeStruct((E, TW), _f32)],
    )(*args)


def _tc_node(pooled2, cnts, v1, c1, v2, c2, an, cn):
    def body(p_r, ct_r, v1_r, c1_r, v2_r, c2_r, an_r, cn2_r, tt_r):
        cnt = ct_r[0, :, 0:1] + ct_r[1, :, 0:1]
        inv = 1.0 / jnp.maximum(cnt, 1.0)
        psum = p_r[0] + p_r[1]
        pooled = (psum[:, :D] + psum[:, D:]) * inv
        h = jnp.maximum(
            jnp.dot(pooled, v1_r[:], preferred_element_type=_f32, precision=lax.Precision.HIGHEST) + c1_r[:], 0.0)
        obj = jnp.maximum(
            jnp.dot(h, v2_r[:], preferred_element_type=_f32, precision=lax.Precision.HIGHEST) + c2_r[:], 0.0)
        tt_r[:] = jnp.concatenate(
            [jnp.dot(obj, an_r[:], preferred_element_type=_f32, precision=lax.Precision.HIGHEST),
             jnp.dot(obj, cn2_r[:], preferred_element_type=_f32, precision=lax.Precision.HIGHEST)], axis=1)

    zero2 = lambda i: (0, 0)
    return pl.pallas_call(
        body,
        grid=(N // BLK_N,),
        in_specs=[pl.BlockSpec((NC, BLK_N, TW), lambda i: (0, i, 0)),
                  pl.BlockSpec((NC, BLK_N, 16), lambda i: (0, i, 0)),
                  pl.BlockSpec((D, D), zero2),
                  pl.BlockSpec((1, D), zero2),
                  pl.BlockSpec((D, D), zero2),
                  pl.BlockSpec((1, D), zero2),
                  pl.BlockSpec((D, D), zero2),
                  pl.BlockSpec((D, D), zero2)],
        out_specs=pl.BlockSpec((BLK_N, TW), lambda i: (i, 0)),
        out_shape=jax.ShapeDtypeStruct((N, TW), _f32),
    )(pooled2, cnts, v1, c1.reshape(1, D), v2, c2.reshape(1, D), an, cn)


def _tc_node_final(pooled2, cnts, v1, c1, v2, c2, w1b, b1b, w2bp, b2bp):
    def body(p_r, ct_r, v1_r, c1_r, v2_r, c2_r, w1_r, bb1_r, w2_r, bb2_r, out_r):
        cnt = ct_r[0, :, 0:1] + ct_r[1, :, 0:1]
        inv = 1.0 / jnp.maximum(cnt, 1.0)
        psum = p_r[0] + p_r[1]
        pooled = (psum[:, :D] + psum[:, D:]) * inv
        h = jnp.maximum(
            jnp.dot(pooled, v1_r[:], preferred_element_type=_f32, precision=lax.Precision.HIGHEST) + c1_r[:], 0.0)
        obj = jnp.maximum(
            jnp.dot(h, v2_r[:], preferred_element_type=_f32, precision=lax.Precision.HIGHEST) + c2_r[:], 0.0)
        hb = jnp.maximum(
            jnp.dot(obj, w1_r[:], preferred_element_type=_f32, precision=lax.Precision.HIGHEST) + bb1_r[:], 0.0)
        out_r[:] = jnp.maximum(
            jnp.dot(hb, w2_r[:], preferred_element_type=_f32, precision=lax.Precision.HIGHEST) + bb2_r[:], 0.0)

    zero2 = lambda i: (0, 0)
    return pl.pallas_call(
        body,
        grid=(N // BLK_N,),
        in_specs=[pl.BlockSpec((NC, BLK_N, TW), lambda i: (0, i, 0)),
                  pl.BlockSpec((NC, BLK_N, 16), lambda i: (0, i, 0)),
                  pl.BlockSpec((D, D), zero2),
                  pl.BlockSpec((1, D), zero2),
                  pl.BlockSpec((D, D), zero2),
                  pl.BlockSpec((1, D), zero2),
                  pl.BlockSpec((D, D), zero2),
                  pl.BlockSpec((1, D), zero2),
                  pl.BlockSpec((D, 128), zero2),
                  pl.BlockSpec((1, 128), zero2)],
        out_specs=pl.BlockSpec((BLK_N, 128), lambda i: (i, 0)),
        out_shape=jax.ShapeDtypeStruct((N, 128), _f32),
    )(pooled2, cnts, v1, c1.reshape(1, D), v2, c2.reshape(1, D),
      w1b, b1b.reshape(1, D), w2bp, b2bp.reshape(1, 128))


def kernel(params, objs, triples):
    s_idx = triples[:, 0]
    p_idx = triples[:, 1]
    o_idx = triples[:, 2]
    objs_pad = jnp.concatenate(
        [objs, jnp.zeros((NPAD - N,), jnp.int32)])

    gc = params["gconv"]
    A = [g["net1"][0][:D] for g in gc]
    B = [g["net1"][0][D:2 * D] for g in gc]
    Cm = [g["net1"][0][2 * D:] for g in gc]
    b1 = [g["net1"][1] for g in gc]
    W2 = [g["net1"][2] for g in gc]
    b2 = [g["net1"][3] for g in gc]

    zeros_cw = jnp.zeros((CH, TW), _f32)
    ones_cw = jnp.ones((CH, TW), _f32)

    t0, tbp = _tc_init(params["obj_emb"], params["pred_emb"],
                       A[0], Cm[0], B[0])
    ttab, p0, cnts_full = _sc_init(
        t0, tbp, objs_pad, p_idx, s_idx, o_idx, zeros_cw, ones_cw)
    cnts = cnts_full[:, :, :16]

    ga, gb = _sc_gather(ttab, s_idx, o_idx)
    ns, npv, no = _tc_edge(ga, gb, p0, None, b1[0], W2[0], b2[0], first=True)
    pooled2 = _sc_scatter(ns, no, s_idx, o_idx, zeros_cw)

    for li in range(1, 5):
        n2 = gc[li - 1]["net2"]
        tt = _tc_node(pooled2, cnts, n2[0], n2[1], n2[2], n2[3],
                      A[li], Cm[li])
        ga, gb = _sc_gather(tt, s_idx, o_idx)
        ns, npv, no = _tc_edge(ga, gb, npv, B[li], b1[li], W2[li], b2[li],
                               first=False)
        pooled2 = _sc_scatter(ns, no, s_idx, o_idx, zeros_cw)

    n2 = gc[4]["net2"]
    bn = params["box_net"]
    w2bp = jnp.zeros((D, 128), _f32).at[:, :4].set(bn[2])
    b2bp = jnp.zeros((128,), _f32).at[:4].set(bn[3])
    boxes_pad = _tc_node_final(pooled2, cnts, n2[0], n2[1], n2[2], n2[3],
                               bn[0], bn[1], w2bp, b2bp)
    return boxes_pad[:, :4]


# speedup vs baseline: 3.0867x; 1.4997x over previous
import functools

import jax
import jax.numpy as jnp
from jax import lax
from jax.experimental import pallas as pl
from jax.experimental.pallas import tpu as pltpu
from jax.experimental.pallas import tpu_sc as plsc

N = 10000
E = 160000
D = 64
TW = 2 * D
NOBJ = 101
NPRED = 46
NC, NSC = 2, 16
NW = NC * NSC
CH = 128
ECH = E // CH
EIT = (ECH + NW - 1) // NW
NPAD = 10112
NCHN = NPAD // CH
NITN = (NCHN + NW - 1) // NW
OUTW = 10
RPW = N // OUTW
STG = 40
NSTG = RPW // STG

BLK_E = 2000
BLK_N = 2000

_MESH = plsc.VectorSubcoreMesh(
    core_axis_name="c", subcore_axis_name="s",
    num_cores=NC, num_subcores=NSC)

_f32 = jnp.float32


@functools.partial(
    pl.kernel,
    out_type=[jax.ShapeDtypeStruct((NPAD, TW), _f32),
              jax.ShapeDtypeStruct((E, TW), _f32),
              jax.ShapeDtypeStruct((NC, N, TW), _f32)],
    mesh=_MESH,
    scratch_types=[pltpu.VMEM((CH,), jnp.int32),
                   pltpu.VMEM((CH,), jnp.int32),
                   pltpu.VMEM((CH, TW), _f32),
                   pltpu.VMEM((CH, TW), _f32),
                   pltpu.VMEM((STG, TW), _f32),
                   pltpu.VMEM_SHARED((N, TW), _f32),
                   pltpu.SemaphoreType.DMA],
)
def _sc_init(t0, tbp, objs_r, p_r, s_r, o_r, zeros_in, ones_in,
             ttab, p0, cnts,
             idx_a, idx_b, rows_a, ones_v, zstg, acc, sem_a):
    cid = lax.axis_index("c")
    sid = lax.axis_index("s")
    wid = sid * NC + cid

    pltpu.sync_copy(ones_in, ones_v)

    @pl.when(sid < OUTW)
    def _zero():
        pltpu.sync_copy(zeros_in.at[pl.ds(0, STG)], zstg)
        for j in range(NSTG):
            pltpu.sync_copy(zstg, acc.at[pl.ds(sid * RPW + j * STG, STG)])

    def node_it(k, carry):
        c = wid + NW * k

        @pl.when(c < NCHN)
        def _():
            base = c * CH
            pltpu.sync_copy(objs_r.at[pl.ds(base, CH)], idx_a)
            pltpu.async_copy(t0.at[idx_a], rows_a, sem_a).wait()
            pltpu.sync_copy(rows_a, ttab.at[pl.ds(base, CH)])
        return carry

    lax.fori_loop(0, NITN, node_it, 0)

    plsc.subcore_barrier()

    def edge_it(k, carry):
        c = wid + NW * k

        @pl.when(c < ECH)
        def _():
            base = c * CH
            pltpu.sync_copy(p_r.at[pl.ds(base, CH)], idx_a)
            pltpu.async_copy(tbp.at[idx_a], rows_a, sem_a).wait()
            pltpu.sync_copy(rows_a, p0.at[pl.ds(base, CH)])
            pltpu.sync_copy(s_r.at[pl.ds(base, CH)], idx_a)
            pltpu.sync_copy(o_r.at[pl.ds(base, CH)], idx_b)
            pltpu.sync_copy(ones_v, acc.at[idx_a], add=True)
            pltpu.sync_copy(ones_v, acc.at[idx_b], add=True)
        return carry

    lax.fori_loop(0, EIT, edge_it, 0)

    plsc.subcore_barrier()

    @pl.when(sid < OUTW)
    def _out():
        for j in range(NSTG):
            pltpu.sync_copy(acc.at[pl.ds(sid * RPW + j * STG, STG)], zstg)
            pltpu.sync_copy(zstg, cnts.at[cid, pl.ds(sid * RPW + j * STG, STG)])


@functools.partial(
    pl.kernel,
    out_type=[jax.ShapeDtypeStruct((E, TW), _f32),
              jax.ShapeDtypeStruct((E, TW), _f32)],
    mesh=_MESH,
    scratch_types=[pltpu.VMEM((CH,), jnp.int32),
                   pltpu.VMEM((CH,), jnp.int32),
                   pltpu.VMEM((CH,), jnp.int32),
                   pltpu.VMEM((CH,), jnp.int32),
                   pltpu.VMEM((CH, TW), _f32),
                   pltpu.VMEM((CH, TW), _f32),
                   pltpu.VMEM((CH, TW), _f32),
                   pltpu.VMEM((CH, TW), _f32),
                   pltpu.SemaphoreType.DMA,
                   pltpu.SemaphoreType.DMA,
                   pltpu.SemaphoreType.DMA,
                   pltpu.SemaphoreType.DMA],
)
def _sc_gather(tt, s_r, o_r, ga, gb,
               ia0, ib0, ia1, ib1, ra0, rb0, ra1, rb1,
               sa0, sb0, sa1, sb1):
    cid = lax.axis_index("c")
    sid = lax.axis_index("s")
    wid = sid * NC + cid
    bufs = ((ia0, ib0, ra0, rb0, sa0, sb0), (ia1, ib1, ra1, rb1, sa1, sb1))

    def start(k, b):
        ia, ib, ra, rb, sa, sb = bufs[b]
        base = (wid + NW * k) * CH
        pltpu.sync_copy(s_r.at[pl.ds(base, CH)], ia)
        pltpu.sync_copy(o_r.at[pl.ds(base, CH)], ib)
        pltpu.async_copy(tt.at[ia], ra, sa)
        pltpu.async_copy(tt.at[ib], rb, sb)

    def finish(k, b):
        ia, ib, ra, rb, sa, sb = bufs[b]
        base = (wid + NW * k) * CH
        pltpu.make_async_copy(tt.at[ia], ra, sa).wait()
        pltpu.make_async_copy(tt.at[ib], rb, sb).wait()
        pltpu.sync_copy(ra, ga.at[pl.ds(base, CH)])
        pltpu.sync_copy(rb, gb.at[pl.ds(base, CH)])

    start(0, 0)

    def body(i, carry):
        k0 = 1 + 2 * i
        start(k0, 1)
        finish(k0 - 1, 0)
        start(k0 + 1, 0)
        finish(k0, 1)
        return carry

    lax.fori_loop(0, 18, body, 0)
    start(37, 1)
    finish(36, 0)
    finish(37, 1)
    start(38, 0)
    finish(38, 0)

    @pl.when(wid < ECH - 39 * NW)
    def _tail():
        start(39, 1)
        finish(39, 1)


CHS = 64
NCHS = E // CHS


@functools.partial(
    pl.kernel,
    out_type=jax.ShapeDtypeStruct((NC, N, TW), _f32),
    mesh=_MESH,
    scratch_types=[pltpu.VMEM((CHS,), jnp.int32),
                   pltpu.VMEM((CHS,), jnp.int32),
                   pltpu.VMEM((CHS,), jnp.int32),
                   pltpu.VMEM((CHS,), jnp.int32),
                   pltpu.VMEM((CHS, TW), _f32),
                   pltpu.VMEM((CHS, TW), _f32),
                   pltpu.VMEM((CHS, TW), _f32),
                   pltpu.VMEM((CHS, TW), _f32),
                   pltpu.VMEM((STG, TW), _f32),
                   pltpu.VMEM_SHARED((N, TW), _f32),
                   pltpu.SemaphoreType.DMA,
                   pltpu.SemaphoreType.DMA,
                   pltpu.SemaphoreType.DMA,
                   pltpu.SemaphoreType.DMA],
)
def _sc_scatter(vs, vo, s_r, o_r, zeros_in, pooled,
                ia0, ib0, ia1, ib1, ra0, rb0, ra1, rb1, zstage, acc,
                sa0, sb0, sa1, sb1):
    cid = lax.axis_index("c")
    sid = lax.axis_index("s")
    wid = sid * NC + cid
    bufs = ((ia0, ib0, ra0, rb0, sa0, sb0), (ia1, ib1, ra1, rb1, sa1, sb1))

    @pl.when(sid < OUTW)
    def _zero():
        pltpu.sync_copy(zeros_in.at[pl.ds(0, STG)], zstage)
        for j in range(NSTG):
            pltpu.sync_copy(zstage, acc.at[pl.ds(sid * RPW + j * STG, STG)])

    plsc.subcore_barrier()

    def start(k, b):
        ia, ib, ra, rb, sa, sb = bufs[b]
        base = (wid + NW * k) * CHS
        pltpu.sync_copy(s_r.at[pl.ds(base, CHS)], ia)
        pltpu.sync_copy(o_r.at[pl.ds(base, CHS)], ib)
        pltpu.async_copy(vs.at[pl.ds(base, CHS)], ra, sa)
        pltpu.async_copy(vo.at[pl.ds(base, CHS)], rb, sb)

    def finish(k, b):
        ia, ib, ra, rb, sa, sb = bufs[b]
        base = (wid + NW * k) * CHS
        pltpu.make_async_copy(vs.at[pl.ds(base, CHS)], ra, sa).wait()
        pltpu.make_async_copy(vo.at[pl.ds(base, CHS)], rb, sb).wait()
        pltpu.sync_copy(ra, acc.at[ia], add=True)
        pltpu.sync_copy(rb, acc.at[ib], add=True)

    start(0, 0)

    def body(i, carry):
        k0 = 1 + 2 * i
        start(k0, 1)
        finish(k0 - 1, 0)
        start(k0 + 1, 0)
        finish(k0, 1)
        return carry

    lax.fori_loop(0, 38, body, 0)
    start(77, 1)
    finish(76, 0)
    finish(77, 1)

    @pl.when(wid < NCHS - 78 * NW)
    def _tail():
        start(78, 0)
        finish(78, 0)

    plsc.subcore_barrier()

    @pl.when(sid < OUTW)
    def _out():
        for j in range(NSTG):
            pltpu.sync_copy(acc.at[pl.ds(sid * RPW + j * STG, STG)], zstage)
            pltpu.sync_copy(zstage, pooled.at[cid, pl.ds(sid * RPW + j * STG, STG)])


def _tc_init(obj_emb, pred_emb):
    def body(oe, pe, t0, tbp):
        t0[:] = jnp.concatenate(
            [oe[:], jnp.zeros((NOBJ, D), _f32)], axis=1)
        tbp[:] = jnp.concatenate(
            [pe[:], jnp.zeros((NPRED, D), _f32)], axis=1)

    return pl.pallas_call(
        body,
        out_shape=[jax.ShapeDtypeStruct((NOBJ, TW), _f32),
                   jax.ShapeDtypeStruct((NPRED, TW), _f32)],
    )(obj_emb, pred_emb)


def _tc_edge(ga, gb, pv, w1, b1, w2, b2, first):
    def body(ga_r, gb_r, pv_r, w1_r, b1_r, w2_r, b2_r, os_r, op_r, oo_r):
        if first:
            pvec = pv_r[:, :D]
        else:
            pvec = pv_r[:]
        x = jnp.concatenate([ga_r[:, :D], pvec, gb_r[:, :D]], axis=1)
        h = jnp.maximum(
            jnp.dot(x, w1_r[:], preferred_element_type=_f32) + b1_r[:], 0.0)
        t = jnp.maximum(
            jnp.dot(h, w2_r[:], preferred_element_type=_f32) + b2_r[:], 0.0)
        z = jnp.zeros((t.shape[0], D), _f32)
        os_r[:] = jnp.concatenate([t[:, :D], z], axis=1)
        op_r[:] = t[:, D:2 * D]
        oo_r[:] = jnp.concatenate([z, t[:, 2 * D:]], axis=1)

    row = lambda i: (i, 0)
    zero = lambda i: (0, 0)
    in_specs = [pl.BlockSpec((BLK_E, TW), row),
                pl.BlockSpec((BLK_E, TW), row),
                pl.BlockSpec((BLK_E, TW if first else D), row),
                pl.BlockSpec((3 * D, D), zero),
                pl.BlockSpec((1, D), zero),
                pl.BlockSpec((D, 3 * D), zero),
                pl.BlockSpec((1, 3 * D), zero)]
    args = [ga, gb, pv, w1, b1.reshape(1, D), w2, b2.reshape(1, 3 * D)]
    return pl.pallas_call(
        body,
        grid=(E // BLK_E,),
        in_specs=in_specs,
        out_specs=[pl.BlockSpec((BLK_E, TW), row),
                   pl.BlockSpec((BLK_E, D), row),
                   pl.BlockSpec((BLK_E, TW), row)],
        out_shape=[jax.ShapeDtypeStruct((E, TW), _f32),
                   jax.ShapeDtypeStruct((E, D), _f32),
                   jax.ShapeDtypeStruct((E, TW), _f32)],
    )(*args)


def _tc_node(pooled2, cnts, v1, c1, v2, c2):
    def body(p_r, ct_r, v1_r, c1_r, v2_r, c2_r, tt_r):
        cnt = ct_r[0, :, 0:1] + ct_r[1, :, 0:1]
        psum = p_r[0] + p_r[1]
        pooled = (psum[:, :D] + psum[:, D:]) / jnp.maximum(cnt, 1.0)
        h = jnp.maximum(
            jnp.dot(pooled, v1_r[:], preferred_element_type=_f32) + c1_r[:], 0.0)
        obj = jnp.maximum(
            jnp.dot(h, v2_r[:], preferred_element_type=_f32) + c2_r[:], 0.0)
        tt_r[:] = jnp.concatenate(
            [obj, jnp.zeros((obj.shape[0], D), _f32)], axis=1)

    zero2 = lambda i: (0, 0)
    return pl.pallas_call(
        body,
        grid=(N // BLK_N,),
        in_specs=[pl.BlockSpec((NC, BLK_N, TW), lambda i: (0, i, 0)),
                  pl.BlockSpec((NC, BLK_N, 16), lambda i: (0, i, 0)),
                  pl.BlockSpec((D, D), zero2),
                  pl.BlockSpec((1, D), zero2),
                  pl.BlockSpec((D, D), zero2),
                  pl.BlockSpec((1, D), zero2)],
        out_specs=pl.BlockSpec((BLK_N, TW), lambda i: (i, 0)),
        out_shape=jax.ShapeDtypeStruct((N, TW), _f32),
    )(pooled2, cnts, v1, c1.reshape(1, D), v2, c2.reshape(1, D))


def _tc_node_final(pooled2, cnts, v1, c1, v2, c2, w1b, b1b, w2bp, b2bp):
    def body(p_r, ct_r, v1_r, c1_r, v2_r, c2_r, w1_r, bb1_r, w2_r, bb2_r, out_r):
        cnt = ct_r[0, :, 0:1] + ct_r[1, :, 0:1]
        psum = p_r[0] + p_r[1]
        pooled = (psum[:, :D] + psum[:, D:]) / jnp.maximum(cnt, 1.0)
        h = jnp.maximum(
            jnp.dot(pooled, v1_r[:], preferred_element_type=_f32) + c1_r[:], 0.0)
        obj = jnp.maximum(
            jnp.dot(h, v2_r[:], preferred_element_type=_f32) + c2_r[:], 0.0)
        hb = jnp.maximum(
            jnp.dot(obj, w1_r[:], preferred_element_type=_f32) + bb1_r[:], 0.0)
        out_r[:] = jnp.maximum(
            jnp.dot(hb, w2_r[:], preferred_element_type=_f32) + bb2_r[:], 0.0)

    zero2 = lambda i: (0, 0)
    return pl.pallas_call(
        body,
        grid=(N // BLK_N,),
        in_specs=[pl.BlockSpec((NC, BLK_N, TW), lambda i: (0, i, 0)),
                  pl.BlockSpec((NC, BLK_N, 16), lambda i: (0, i, 0)),
                  pl.BlockSpec((D, D), zero2),
                  pl.BlockSpec((1, D), zero2),
                  pl.BlockSpec((D, D), zero2),
                  pl.BlockSpec((1, D), zero2),
                  pl.BlockSpec((D, D), zero2),
                  pl.BlockSpec((1, D), zero2),
                  pl.BlockSpec((D, 128), zero2),
                  pl.BlockSpec((1, 128), zero2)],
        out_specs=pl.BlockSpec((BLK_N, 128), lambda i: (i, 0)),
        out_shape=jax.ShapeDtypeStruct((N, 128), _f32),
    )(pooled2, cnts, v1, c1.reshape(1, D), v2, c2.reshape(1, D),
      w1b, b1b.reshape(1, D), w2bp, b2bp.reshape(1, 128))


def kernel(params, objs, triples):
    s_idx = triples[:, 0]
    p_idx = triples[:, 1]
    o_idx = triples[:, 2]
    objs_pad = jnp.concatenate(
        [objs, jnp.zeros((NPAD - N,), jnp.int32)])

    gc = params["gconv"]
    W1 = [g["net1"][0] for g in gc]
    b1 = [g["net1"][1] for g in gc]
    W2 = [g["net1"][2] for g in gc]
    b2 = [g["net1"][3] for g in gc]

    zeros_cw = jnp.zeros((CH, TW), _f32)
    ones_cw = jnp.ones((CH, TW), _f32)

    t0, tbp = _tc_init(params["obj_emb"], params["pred_emb"])
    ttab, p0, cnts_full = _sc_init(
        t0, tbp, objs_pad, p_idx, s_idx, o_idx, zeros_cw, ones_cw)
    cnts = cnts_full[:, :, :16]

    ga, gb = _sc_gather(ttab, s_idx, o_idx)
    ns, npv, no = _tc_edge(ga, gb, p0, W1[0], b1[0], W2[0], b2[0], first=True)
    pooled2 = _sc_scatter(ns, no, s_idx, o_idx, zeros_cw)

    for li in range(1, 5):
        n2 = gc[li - 1]["net2"]
        tt = _tc_node(pooled2, cnts, n2[0], n2[1], n2[2], n2[3])
        ga, gb = _sc_gather(tt, s_idx, o_idx)
        ns, npv, no = _tc_edge(ga, gb, npv, W1[li], b1[li], W2[li], b2[li],
                               first=False)
        pooled2 = _sc_scatter(ns, no, s_idx, o_idx, zeros_cw)

    n2 = gc[4]["net2"]
    bn = params["box_net"]
    w2bp = jnp.zeros((D, 128), _f32).at[:, :4].set(bn[2])
    b2bp = jnp.zeros((128,), _f32).at[:4].set(bn[3])
    boxes_pad = _tc_node_final(pooled2, cnts, n2[0], n2[1], n2[2], n2[3],
                               bn[0], bn[1], w2bp, b2bp)
    return boxes_pad[:, :4]
